# Initial kernel scaffold; baseline (speedup 1.0000x reference)
import jax
import jax.numpy as jnp
from jax.experimental import pallas as pl


def kernel(positions, types, object_positions, object_colors, object_shapes, object_materials, object_sizes, question, q_emb, color_emb, shape_emb, material_emb, size_emb, W, b):
    raise NotImplementedError("write your pallas kernel here")


# speedup vs baseline: 3.4138x; 3.4138x over previous
import functools

import jax
import jax.numpy as jnp
from jax import lax
from jax.experimental import pallas as pl
from jax.experimental.pallas import tpu as pltpu
from jax.experimental.pallas import tpu_sc as plsc

B = 16384
L = 20
EMB = 128
HID = 512
R = B * 10


_NC, _NS = 2, 16
_NW = _NC * _NS
_Q = B * L
_PER_W = _Q // _NW
_CHUNK = 128
_NCHUNK = _PER_W // _CHUNK


def _qgather_body(table_hbm, idx_hbm, out_hbm, idx_v, rows_v, gsem, ssem):
    wid = lax.axis_index("s") * _NC + lax.axis_index("c")
    base = wid * _PER_W
    pltpu.sync_copy(idx_hbm.at[wid], idx_v)

    def body_sync(i, carry):
        pltpu.async_copy(table_hbm.at[idx_v.at[i]], rows_v, gsem).wait()
        pltpu.async_copy(rows_v,
                         out_hbm.at[pl.ds(base + i * _CHUNK, _CHUNK)],
                         ssem).wait()
        return carry

    lax.fori_loop(0, _NCHUNK, body_sync, 0)


def _questions_gather(q_emb, question_flat):
    mesh = plsc.VectorSubcoreMesh(core_axis_name="c", subcore_axis_name="s")
    k = pl.kernel(
        _qgather_body,
        out_type=jax.ShapeDtypeStruct((_Q, EMB), jnp.float32),
        mesh=mesh,
        scratch_types=[
            pltpu.VMEM((_NCHUNK, _CHUNK), jnp.int32),
            pltpu.VMEM((_CHUNK, EMB), jnp.float32),
            pltpu.SemaphoreType.DMA,
            pltpu.SemaphoreType.DMA,
        ],
    )
    return k(q_emb, question_flat.reshape(_NW, _NCHUNK, _CHUNK))


def _proj_body(ce_ref, se_ref, me_ref, ze_ref, wc_ref, ws_ref, wm_ref,
               wz_ref, posb_ref, t_ref):
    tc = jnp.dot(ce_ref[...], wc_ref[...], preferred_element_type=jnp.float32)
    ts = jnp.dot(se_ref[...], ws_ref[...], preferred_element_type=jnp.float32)
    tm = jnp.dot(me_ref[...], wm_ref[...], preferred_element_type=jnp.float32)
    tz = jnp.dot(ze_ref[...], wz_ref[...], preferred_element_type=jnp.float32)
    zero = jnp.zeros((56, HID), jnp.float32)
    t_ref[...] = jnp.concatenate([tc, ts, tm, tz, posb_ref[...], zero],
                                 axis=0)


def _build_table(color_emb, shape_emb, material_emb, size_emb, W, b):
    ce = jnp.zeros((16, EMB), jnp.float32).at[:9].set(color_emb)
    se = jnp.zeros((16, EMB), jnp.float32).at[:4].set(shape_emb)
    me = jnp.zeros((16, EMB), jnp.float32).at[:3].set(material_emb)
    ze = jnp.zeros((16, EMB), jnp.float32).at[:3].set(size_emb)
    wc = W[:, 3:131].T
    ws = W[:, 131:259].T
    wm = W[:, 259:387].T
    wz = W[:, 387:515].T
    posb = jnp.zeros((8, HID), jnp.float32).at[:3].set(W[:, :3].T).at[3].set(b)
    return pl.pallas_call(
        _proj_body,
        out_shape=jax.ShapeDtypeStruct((128, HID), jnp.float32),
    )(ce, se, me, ze, wc, ws, wm, wz, posb)


_RBLK = 2048


def _ore_body(c_ref, s_ref, m_ref, z_ref, p_ref, t_ref, o_ref):
    col = lax.broadcasted_iota(jnp.int32, (_RBLK, 128), 1)
    hot = ((col == c_ref[...]) | (col == 16 + s_ref[...])
           | (col == 32 + m_ref[...]) | (col == 48 + z_ref[...])
           | (col == 67))
    p = p_ref[...]
    x = jnp.where(hot, 1.0, 0.0)
    x = x + jnp.where(col == 64, p[:, 0:1], 0.0)
    x = x + jnp.where(col == 65, p[:, 1:2], 0.0)
    x = x + jnp.where(col == 66, p[:, 2:3], 0.0)
    o_ref[...] = jnp.dot(x, t_ref[...], preferred_element_type=jnp.float32)


def _ore_compute(colors, shapes, materials, sizes, positions, table):
    c = colors.reshape(R, 1)
    s = shapes.reshape(R, 1)
    m = materials.reshape(R, 1)
    z = sizes.reshape(R, 1)
    p = positions.reshape(R, 3)
    grid = R // _RBLK
    idx = lambda i: (i, 0)
    bs1 = pl.BlockSpec((_RBLK, 1), idx)
    return pl.pallas_call(
        _ore_body,
        grid=(grid,),
        in_specs=[bs1, bs1, bs1, bs1,
                  pl.BlockSpec((_RBLK, 3), idx),
                  pl.BlockSpec((128, HID), lambda i: (0, 0))],
        out_specs=pl.BlockSpec((_RBLK, HID), idx),
        out_shape=jax.ShapeDtypeStruct((R, HID), jnp.float32),
    )(c, s, m, z, p, table)


_MBLK = 2048


def _mask_body(t_ref, m_ref):
    col = lax.broadcasted_iota(jnp.int32, (_MBLK, 30), 1)
    t = t_ref[...]
    m_ref[...] = jnp.where(col < 10, (t == 1).astype(jnp.float32),
                           (t == 2).astype(jnp.float32))


def _masks(types):
    idx = lambda i: (i, 0)
    return pl.pallas_call(
        _mask_body,
        grid=(B // _MBLK,),
        in_specs=[pl.BlockSpec((_MBLK, 30), idx)],
        out_specs=pl.BlockSpec((_MBLK, 30), idx),
        out_shape=jax.ShapeDtypeStruct((B, 30), jnp.float32),
    )(types)


def kernel(positions, types, object_positions, object_colors, object_shapes,
           object_materials, object_sizes, question, q_emb, color_emb,
           shape_emb, material_emb, size_emb, W, b):
    table = _build_table(color_emb, shape_emb, material_emb, size_emb, W, b)
    ore = _ore_compute(object_colors, object_shapes, object_materials,
                       object_sizes, object_positions, table)
    questions = _questions_gather(q_emb, question)
    m = _masks(types)
    object_mask = m[:, :10].reshape(B, 1, 1, 10)
    question_mask = m[:, 10:].reshape(B, 1, 1, 20)
    mixed_mask = m.reshape(B, 1, 1, 30)
    return (ore.reshape(B, 10, HID), questions.reshape(B, L, EMB),
            object_mask, question_mask, mixed_mask)
